# Initial kernel scaffold; baseline (speedup 1.0000x reference)
import jax
import jax.numpy as jnp
from jax.experimental import pallas as pl


def kernel(x, edge_index, params):
    raise NotImplementedError("write your pallas kernel here")


# speedup vs baseline: 5.3265x; 5.3265x over previous
import functools

import jax
import jax.numpy as jnp
from jax import lax
from jax.experimental import pallas as pl
from jax.experimental.pallas import tpu as pltpu
from jax.experimental.pallas import tpu_sc as plsc

N_NODES = 10000
N_PAD = 10240
DUMMY = N_NODES
NT = 32
CHUNK = 128
IDX_BLK = 16
J_STEPS = 160
ROWS_PER_TILE = N_PAD // 16
RB = 256
EPS = 1e-5
_f32 = jnp.float32


@functools.lru_cache(maxsize=None)
def _make_propagate(w):
  mesh = plsc.VectorSubcoreMesh(core_axis_name="c", subcore_axis_name="s")

  def body(table_h, src_h, dst_h, zeros_h, out_h, src_v, dst_v, rows_v, sem,
           acc_sh):
    cid = lax.axis_index("c")
    sid = lax.axis_index("s")
    wid = cid * 16 + sid
    pltpu.sync_copy(zeros_h, acc_sh.at[pl.ds(sid * ROWS_PER_TILE,
                                             ROWS_PER_TILE)])
    plsc.subcore_barrier()

    def outer(o, carry):
      pltpu.sync_copy(src_h.at[wid].at[pl.ds(o * IDX_BLK, IDX_BLK)], src_v)
      pltpu.sync_copy(dst_h.at[wid].at[pl.ds(o * IDX_BLK, IDX_BLK)], dst_v)
      for b in range(IDX_BLK):
        pltpu.async_copy(table_h.at[src_v.at[b]], rows_v, sem).wait()
        pltpu.sync_copy(rows_v, acc_sh.at[dst_v.at[b]], add=True)
      return carry

    lax.fori_loop(0, J_STEPS // IDX_BLK, outer, 0)
    plsc.subcore_barrier()
    pltpu.sync_copy(
        acc_sh.at[pl.ds(sid * ROWS_PER_TILE, ROWS_PER_TILE)],
        out_h.at[cid].at[pl.ds(sid * ROWS_PER_TILE, ROWS_PER_TILE)])

  return pl.kernel(
      body,
      out_type=jax.ShapeDtypeStruct((2, N_PAD, w), _f32),
      mesh=mesh,
      scratch_types=[
          pltpu.VMEM((IDX_BLK, CHUNK), jnp.int32),
          pltpu.VMEM((IDX_BLK, CHUNK), jnp.int32),
          pltpu.VMEM((CHUNK, w), _f32),
          pltpu.SemaphoreType.DMA,
          pltpu.VMEM_SHARED((N_PAD, w), _f32),
      ],
      compiler_params=pltpu.CompilerParams(use_tc_tiling_on_sc=False),
  )


def _propagate(table, src, dst):
  w = table.shape[1]
  zeros = jnp.zeros((ROWS_PER_TILE, w), _f32)
  return _make_propagate(w)(table, src, dst, zeros)


def _pre_kernel(*refs, has_act, has_w, scale_s, cw, nc):
  it = iter(refs)
  h = next(it)[...]
  if has_act:
    mean, sv, g, be = (next(it)[...] for _ in range(4))
    h = jnp.maximum((h - mean) / sv * g + be, 0.0)
  if has_w:
    o = jnp.dot(h, next(it)[...], preferred_element_type=_f32)
  else:
    o = h
  if scale_s:
    o = o * next(it)[...]
  out_refs = list(it)
  for k in range(nc):
    out_refs[k][...] = o[:, k * cw:(k + 1) * cw]


def _run_pre(hin, act, w_mat, s_col, cw, scale_s=True):
  has_act = act is not None
  has_w = w_mat is not None
  din = hin.shape[1]
  dout = w_mat.shape[1] if has_w else din
  nc = dout // cw
  grid = N_PAD // RB
  in_specs = [pl.BlockSpec((RB, din), lambda i: (i, 0))]
  args = [hin]
  if has_act:
    in_specs += [pl.BlockSpec((1, din), lambda i: (0, 0))] * 4
    args += [v.reshape(1, din) for v in act]
  if has_w:
    in_specs.append(pl.BlockSpec((din, dout), lambda i: (0, 0)))
    args.append(w_mat)
  if scale_s:
    in_specs.append(pl.BlockSpec((RB, 1), lambda i: (i, 0)))
    args.append(s_col)
  fn = pl.pallas_call(
      functools.partial(_pre_kernel, has_act=has_act, has_w=has_w,
                        scale_s=scale_s, cw=cw, nc=nc),
      grid=(grid,),
      in_specs=in_specs,
      out_specs=[pl.BlockSpec((RB, cw), lambda i: (i, 0))] * nc,
      out_shape=[jax.ShapeDtypeStruct((N_PAD, cw), _f32)] * nc,
  )
  return fn(*args)


def _post_kernel(*refs, nc, cw):
  p_refs = refs[:nc]
  s_ref, b_ref, agg_ref, sum_ref = refs[nc:]
  i = pl.program_id(0)
  p = jnp.concatenate([r[0] + r[1] for r in p_refs], axis=1)
  o = p * s_ref[...] + b_ref[...]
  agg_ref[...] = o
  rows = i * RB + lax.broadcasted_iota(jnp.int32, (RB, 1), 0)
  om = jnp.where(rows < N_NODES, o, 0.0)
  ps = jnp.sum(om, axis=0, keepdims=True)
  upd = jnp.concatenate([ps, jnp.zeros((7, om.shape[1]), _f32)], axis=0)

  @pl.when(i == 0)
  def _():
    sum_ref[...] = jnp.zeros_like(sum_ref)

  sum_ref[...] += upd


def _run_post(p_parts, s_col, bias):
  nc = len(p_parts)
  cw = p_parts[0].shape[2]
  dout = nc * cw
  grid = N_PAD // RB
  in_specs = [pl.BlockSpec((2, RB, cw), lambda i: (0, i, 0))] * nc
  args = list(p_parts)
  in_specs.append(pl.BlockSpec((RB, 1), lambda i: (i, 0)))
  args.append(s_col)
  in_specs.append(pl.BlockSpec((1, dout), lambda i: (0, 0)))
  args.append(bias.reshape(1, dout))
  fn = pl.pallas_call(
      functools.partial(_post_kernel, nc=nc, cw=cw),
      grid=(grid,),
      in_specs=in_specs,
      out_specs=[
          pl.BlockSpec((RB, dout), lambda i: (i, 0)),
          pl.BlockSpec((8, dout), lambda i: (0, 0)),
      ],
      out_shape=[
          jax.ShapeDtypeStruct((N_PAD, dout), _f32),
          jax.ShapeDtypeStruct((8, dout), _f32),
      ],
  )
  return fn(*args)


def _var_kernel(agg_ref, mean_ref, out_ref):
  i = pl.program_id(0)
  d = agg_ref[...] - mean_ref[...]
  rows = i * RB + lax.broadcasted_iota(jnp.int32, (RB, 1), 0)
  d = jnp.where(rows < N_NODES, d, 0.0)
  ps = jnp.sum(d * d, axis=0, keepdims=True)
  upd = jnp.concatenate([ps, jnp.zeros((7, d.shape[1]), _f32)], axis=0)

  @pl.when(i == 0)
  def _():
    out_ref[...] = jnp.zeros_like(out_ref)

  out_ref[...] += upd


def _run_var(agg, mean):
  dout = agg.shape[1]
  fn = pl.pallas_call(
      _var_kernel,
      grid=(N_PAD // RB,),
      in_specs=[
          pl.BlockSpec((RB, dout), lambda i: (i, 0)),
          pl.BlockSpec((1, dout), lambda i: (0, 0)),
      ],
      out_specs=pl.BlockSpec((8, dout), lambda i: (0, 0)),
      out_shape=jax.ShapeDtypeStruct((8, dout), _f32),
  )
  return fn(agg, mean.reshape(1, dout))


def _bn_stats(agg, sums, g, be):
  mean = sums[0] / N_NODES
  var = _run_var(agg, mean)[0] / N_NODES
  sv = jnp.sqrt(var + EPS)
  return mean, sv, g, be


def kernel(x, edge_index, params):
  ei = edge_index.astype(jnp.int32)
  loops = jnp.arange(N_NODES, dtype=jnp.int32)
  e_pad = NT * J_STEPS * CHUNK
  fill = jnp.full((e_pad - ei.shape[1] - N_NODES,), DUMMY, jnp.int32)
  src = jnp.concatenate([ei[0], loops, fill]).reshape(NT, J_STEPS, CHUNK)
  dst = jnp.concatenate([ei[1], loops, fill]).reshape(NT, J_STEPS, CHUNK)

  degp = _propagate(jnp.ones((N_PAD, 16), _f32), src, dst)
  deg = degp[0, :, 0] + degp[1, :, 0]
  mask = jnp.arange(N_PAD) < N_NODES
  s_col = jnp.where(mask, 1.0 / jnp.sqrt(jnp.maximum(deg, 1.0)),
                    0.0).astype(_f32).reshape(N_PAD, 1)

  xp = jnp.zeros((N_PAD, 128), _f32).at[:N_NODES].set(x)

  dims = [(128, 64), (64, 64), (64, 64), (64, 128), (128, 1024), (1024, 512),
          (512, 256), (256, 40)]
  h = xp
  act = None
  for i, (din, dout) in enumerate(dims):
    W = params[f"W{i}"]
    b = params[f"b{i}"]
    g = params[f"g{i}"]
    be = params[f"be{i}"]
    if dout < 64:
      padc = 64 - dout
      W = jnp.pad(W, ((0, 0), (0, padc)))
      b = jnp.pad(b, (0, padc))
      g = jnp.pad(g, (0, padc), constant_values=1.0)
      be = jnp.pad(be, (0, padc))
      dout = 64
    cw = min(dout, 128)
    chunks = _run_pre(h, act, W, s_col, cw)
    parts = [_propagate(c, src, dst) for c in chunks]
    agg, sums = _run_post(parts, s_col, b)
    act = _bn_stats(agg, sums, g, be)
    h = agg

  y = _run_pre(h, act, None, None, h.shape[1], scale_s=False)[0]
  return y[:N_NODES, :40]


# speedup vs baseline: 6.1279x; 1.1504x over previous
import functools

import jax
import jax.numpy as jnp
from jax import lax
from jax.experimental import pallas as pl
from jax.experimental.pallas import tpu as pltpu
from jax.experimental.pallas import tpu_sc as plsc

N_NODES = 10000
N_PAD = 10240
DUMMY = N_NODES
NT = 32
CHUNK = 128
IDX_BLK = 16
J_STEPS = 160
ROWS_PER_TILE = N_PAD // 16
RB = 256
EPS = 1e-5
_f32 = jnp.float32


@functools.lru_cache(maxsize=None)
def _make_propagate(w):
  mesh = plsc.VectorSubcoreMesh(core_axis_name="c", subcore_axis_name="s")

  def body(table_h, src_h, dst_h, zeros_h, out_h, src_v, dst_v, rows_v, sem,
           acc_sh):
    cid = lax.axis_index("c")
    sid = lax.axis_index("s")
    wid = cid * 16 + sid
    pltpu.sync_copy(zeros_h, acc_sh.at[pl.ds(sid * ROWS_PER_TILE,
                                             ROWS_PER_TILE)])
    plsc.subcore_barrier()

    def outer(o, carry):
      pltpu.sync_copy(src_h.at[wid].at[pl.ds(o * IDX_BLK, IDX_BLK)], src_v)
      pltpu.sync_copy(dst_h.at[wid].at[pl.ds(o * IDX_BLK, IDX_BLK)], dst_v)
      pending = pltpu.async_copy(table_h.at[src_v.at[0]], rows_v.at[0], sem)
      for b in range(IDX_BLK):
        pending.wait()
        if b + 1 < IDX_BLK:
          pending = pltpu.async_copy(table_h.at[src_v.at[b + 1]],
                                     rows_v.at[(b + 1) % 2], sem)
        pltpu.sync_copy(rows_v.at[b % 2], acc_sh.at[dst_v.at[b]], add=True)
      return carry

    lax.fori_loop(0, J_STEPS // IDX_BLK, outer, 0)
    plsc.subcore_barrier()
    pltpu.sync_copy(
        acc_sh.at[pl.ds(sid * ROWS_PER_TILE, ROWS_PER_TILE)],
        out_h.at[cid].at[pl.ds(sid * ROWS_PER_TILE, ROWS_PER_TILE)])

  return pl.kernel(
      body,
      out_type=jax.ShapeDtypeStruct((2, N_PAD, w), _f32),
      mesh=mesh,
      scratch_types=[
          pltpu.VMEM((IDX_BLK, CHUNK), jnp.int32),
          pltpu.VMEM((IDX_BLK, CHUNK), jnp.int32),
          pltpu.VMEM((2, CHUNK, w), _f32),
          pltpu.SemaphoreType.DMA,
          pltpu.VMEM_SHARED((N_PAD, w), _f32),
      ],
      compiler_params=pltpu.CompilerParams(use_tc_tiling_on_sc=False),
  )


def _propagate(table, src, dst):
  w = table.shape[1]
  zeros = jnp.zeros((ROWS_PER_TILE, w), _f32)
  return _make_propagate(w)(table, src, dst, zeros)


def _pre_kernel(*refs, has_act, has_w, scale_s, cw, nc):
  it = iter(refs)
  h = next(it)[...]
  if has_act:
    mean, sv, g, be = (next(it)[...] for _ in range(4))
    h = jnp.maximum((h - mean) / sv * g + be, 0.0)
  if has_w:
    o = jnp.dot(h, next(it)[...], preferred_element_type=_f32)
  else:
    o = h
  if scale_s:
    o = o * next(it)[...]
  out_refs = list(it)
  for k in range(nc):
    out_refs[k][...] = o[:, k * cw:(k + 1) * cw]


def _run_pre(hin, act, w_mat, s_col, cw, scale_s=True):
  has_act = act is not None
  has_w = w_mat is not None
  din = hin.shape[1]
  dout = w_mat.shape[1] if has_w else din
  nc = dout // cw
  grid = N_PAD // RB
  in_specs = [pl.BlockSpec((RB, din), lambda i: (i, 0))]
  args = [hin]
  if has_act:
    in_specs += [pl.BlockSpec((1, din), lambda i: (0, 0))] * 4
    args += [v.reshape(1, din) for v in act]
  if has_w:
    in_specs.append(pl.BlockSpec((din, dout), lambda i: (0, 0)))
    args.append(w_mat)
  if scale_s:
    in_specs.append(pl.BlockSpec((RB, 1), lambda i: (i, 0)))
    args.append(s_col)
  fn = pl.pallas_call(
      functools.partial(_pre_kernel, has_act=has_act, has_w=has_w,
                        scale_s=scale_s, cw=cw, nc=nc),
      grid=(grid,),
      in_specs=in_specs,
      out_specs=[pl.BlockSpec((RB, cw), lambda i: (i, 0))] * nc,
      out_shape=[jax.ShapeDtypeStruct((N_PAD, cw), _f32)] * nc,
  )
  return fn(*args)


def _post_kernel(*refs, nc, cw):
  p_refs = refs[:nc]
  s_ref, b_ref, agg_ref, sum_ref = refs[nc:]
  i = pl.program_id(0)
  p = jnp.concatenate([r[0] + r[1] for r in p_refs], axis=1)
  o = p * s_ref[...] + b_ref[...]
  agg_ref[...] = o
  rows = i * RB + lax.broadcasted_iota(jnp.int32, (RB, 1), 0)
  om = jnp.where(rows < N_NODES, o, 0.0)
  ps = jnp.sum(om, axis=0, keepdims=True)
  upd = jnp.concatenate([ps, jnp.zeros((7, om.shape[1]), _f32)], axis=0)

  @pl.when(i == 0)
  def _():
    sum_ref[...] = jnp.zeros_like(sum_ref)

  sum_ref[...] += upd


def _run_post(p_parts, s_col, bias):
  nc = len(p_parts)
  cw = p_parts[0].shape[2]
  dout = nc * cw
  grid = N_PAD // RB
  in_specs = [pl.BlockSpec((2, RB, cw), lambda i: (0, i, 0))] * nc
  args = list(p_parts)
  in_specs.append(pl.BlockSpec((RB, 1), lambda i: (i, 0)))
  args.append(s_col)
  in_specs.append(pl.BlockSpec((1, dout), lambda i: (0, 0)))
  args.append(bias.reshape(1, dout))
  fn = pl.pallas_call(
      functools.partial(_post_kernel, nc=nc, cw=cw),
      grid=(grid,),
      in_specs=in_specs,
      out_specs=[
          pl.BlockSpec((RB, dout), lambda i: (i, 0)),
          pl.BlockSpec((8, dout), lambda i: (0, 0)),
      ],
      out_shape=[
          jax.ShapeDtypeStruct((N_PAD, dout), _f32),
          jax.ShapeDtypeStruct((8, dout), _f32),
      ],
  )
  return fn(*args)


def _var_kernel(agg_ref, mean_ref, out_ref):
  i = pl.program_id(0)
  d = agg_ref[...] - mean_ref[...]
  rows = i * RB + lax.broadcasted_iota(jnp.int32, (RB, 1), 0)
  d = jnp.where(rows < N_NODES, d, 0.0)
  ps = jnp.sum(d * d, axis=0, keepdims=True)
  upd = jnp.concatenate([ps, jnp.zeros((7, d.shape[1]), _f32)], axis=0)

  @pl.when(i == 0)
  def _():
    out_ref[...] = jnp.zeros_like(out_ref)

  out_ref[...] += upd


def _run_var(agg, mean):
  dout = agg.shape[1]
  fn = pl.pallas_call(
      _var_kernel,
      grid=(N_PAD // RB,),
      in_specs=[
          pl.BlockSpec((RB, dout), lambda i: (i, 0)),
          pl.BlockSpec((1, dout), lambda i: (0, 0)),
      ],
      out_specs=pl.BlockSpec((8, dout), lambda i: (0, 0)),
      out_shape=jax.ShapeDtypeStruct((8, dout), _f32),
  )
  return fn(agg, mean.reshape(1, dout))


def _bn_stats(agg, sums, g, be):
  mean = sums[0] / N_NODES
  var = _run_var(agg, mean)[0] / N_NODES
  sv = jnp.sqrt(var + EPS)
  return mean, sv, g, be


def kernel(x, edge_index, params):
  ei = edge_index.astype(jnp.int32)
  loops = jnp.arange(N_NODES, dtype=jnp.int32)
  e_pad = NT * J_STEPS * CHUNK
  fill = jnp.full((e_pad - ei.shape[1] - N_NODES,), DUMMY, jnp.int32)
  src = jnp.concatenate([ei[0], loops, fill]).reshape(NT, J_STEPS, CHUNK)
  dst = jnp.concatenate([ei[1], loops, fill]).reshape(NT, J_STEPS, CHUNK)

  degp = _propagate(jnp.ones((N_PAD, 16), _f32), src, dst)
  deg = degp[0, :, 0] + degp[1, :, 0]
  mask = jnp.arange(N_PAD) < N_NODES
  s_col = jnp.where(mask, 1.0 / jnp.sqrt(jnp.maximum(deg, 1.0)),
                    0.0).astype(_f32).reshape(N_PAD, 1)

  xp = jnp.zeros((N_PAD, 128), _f32).at[:N_NODES].set(x)

  dims = [(128, 64), (64, 64), (64, 64), (64, 128), (128, 1024), (1024, 512),
          (512, 256), (256, 40)]
  h = xp
  act = None
  for i, (din, dout) in enumerate(dims):
    W = params[f"W{i}"]
    b = params[f"b{i}"]
    g = params[f"g{i}"]
    be = params[f"be{i}"]
    if dout < 64:
      padc = 64 - dout
      W = jnp.pad(W, ((0, 0), (0, padc)))
      b = jnp.pad(b, (0, padc))
      g = jnp.pad(g, (0, padc), constant_values=1.0)
      be = jnp.pad(be, (0, padc))
      dout = 64
    cw = min(dout, 128)
    chunks = _run_pre(h, act, W, s_col, cw)
    parts = [_propagate(c, src, dst) for c in chunks]
    agg, sums = _run_post(parts, s_col, b)
    act = _bn_stats(agg, sums, g, be)
    h = agg

  y = _run_pre(h, act, None, None, h.shape[1], scale_s=False)[0]
  return y[:N_NODES, :40]
